# Initial kernel scaffold; baseline (speedup 1.0000x reference)
import jax
import jax.numpy as jnp
from jax.experimental import pallas as pl


def kernel(x, edge_index, batch, Vw, Vb, Aw, Ab, Rw, Rb, gamma, beta, Pw, Pb):
    raise NotImplementedError("write your pallas kernel here")


# speedup vs baseline: 6.5467x; 6.5467x over previous
import functools

import jax
import jax.numpy as jnp
from jax import lax
from jax.experimental import pallas as pl
from jax.experimental.pallas import tpu as pltpu
from jax.experimental.pallas import tpu_sc as plsc

N = 10000
E = 320000
H = 128
G = 100
L = 2

NC = 2
NS = 16
NW = NC * NS
EPW = E // NW
CH = 80
NCH = EPW // CH
NP = 10240
RPS = NP // NS


def _sc_aggr_body(h_hbm, src_hbm, dst_hbm, z_hbm, out_hbm,
                  sidx_v, didx_v, rows_v, shared_aggr, sem):
    c = lax.axis_index("c")
    s = lax.axis_index("s")
    wid = s * NC + c

    pltpu.sync_copy(z_hbm, shared_aggr.at[pl.ds(s * RPS, RPS)])

    pltpu.sync_copy(src_hbm.at[wid], sidx_v)
    pltpu.sync_copy(dst_hbm.at[wid], didx_v)
    plsc.subcore_barrier()

    def step(j, carry):
        pltpu.async_copy(h_hbm.at[sidx_v.at[j]], rows_v, sem).wait()
        pltpu.sync_copy(rows_v, shared_aggr.at[didx_v.at[j]], add=True)
        return carry

    lax.fori_loop(0, NCH, step, 0)
    plsc.subcore_barrier()

    pltpu.sync_copy(shared_aggr.at[pl.ds(s * RPS, RPS)],
                    out_hbm.at[c, pl.ds(s * RPS, RPS)])


def _sc_aggregate(h, src_r, dst_r, zeros):
    mesh = plsc.VectorSubcoreMesh(core_axis_name="c", subcore_axis_name="s")
    return pl.kernel(
        _sc_aggr_body,
        out_type=jax.ShapeDtypeStruct((NC, NP, H), jnp.float32),
        mesh=mesh,
        scratch_types=[
            pltpu.VMEM((NCH, CH), jnp.int32),
            pltpu.VMEM((NCH, CH), jnp.int32),
            pltpu.VMEM((CH, H), jnp.float32),
            pltpu.VMEM_SHARED((NP, H), jnp.float32),
            pltpu.SemaphoreType.DMA,
        ],
        name="sc_edge_aggregate",
    )(h, src_r, dst_r, zeros)


def _combine_body(h_ref, p_ref, b_ref, vw_ref, vb_ref, aw_ref, ab_ref,
                  rw_ref, rb_ref, g_ref, bt_ref, *rest):
    if len(rest) == 3:
        pw_ref, pb_ref, o_ref = rest
        maybe_head = (pw_ref, pb_ref)
    else:
        (o_ref,) = rest
        maybe_head = None
    h = h_ref[:]
    aggr = p_ref[0, :N] + p_ref[1, :N]
    onehot = (b_ref[:] == lax.broadcasted_iota(jnp.int32, (1, H), 1)
              ).astype(jnp.float32)
    pooled = lax.dot_general(onehot, h, (((0,), (0,)), ((), ())),
                             preferred_element_type=jnp.float32)
    r_rot = jnp.dot(pooled, rw_ref[:], preferred_element_type=jnp.float32)
    r_term = jnp.dot(onehot, r_rot, preferred_element_type=jnp.float32)
    hn = (jnp.dot(h, vw_ref[:], preferred_element_type=jnp.float32)
          + jnp.dot(aggr, aw_ref[:], preferred_element_type=jnp.float32)
          + r_term + vb_ref[:] + ab_ref[:] + rb_ref[:])
    hn = jnp.maximum(hn, 0.0)
    mean = jnp.mean(hn, axis=0, keepdims=True)
    cen = hn - mean
    var = jnp.mean(cen * cen, axis=0, keepdims=True)
    out = cen * lax.rsqrt(var + 1e-5) * g_ref[:] + bt_ref[:]
    if maybe_head:
        pw_ref, pb_ref = maybe_head
        out = jnp.dot(out, pw_ref[:], preferred_element_type=jnp.float32) \
            + pb_ref[:]
    o_ref[:] = out


def _combine(h, parts, batch_col, vw, vb, aw, ab, rw, rb, g, bt,
             head=None):
    args = [h, parts, batch_col, vw, vb, aw, ab, rw, rb, g, bt]
    if head is not None:
        args += [head[0], head[1]]
    return pl.pallas_call(
        _combine_body,
        out_shape=jax.ShapeDtypeStruct((N, H), jnp.float32),
        name="tc_combine",
    )(*args)


@jax.jit
def kernel(x, edge_index, batch, Vw, Vb, Aw, Ab, Rw, Rb, gamma, beta, Pw, Pb):
    src_r = edge_index[0].reshape(NW, NCH, CH)
    dst_r = edge_index[1].reshape(NW, NCH, CH)
    zeros = jnp.zeros((RPS, H), dtype=jnp.float32)
    batch_col = batch.reshape(N, 1)

    h = x
    for l in range(L):
        parts = _sc_aggregate(h, src_r, dst_r, zeros)
        head = (Pw, Pb) if l == L - 1 else None
        h = _combine(h, parts, batch_col,
                     Vw[l], Vb[l].reshape(1, H), Aw[l], Ab[l].reshape(1, H),
                     Rw[l], Rb[l].reshape(1, H),
                     gamma[l].reshape(1, H), beta[l].reshape(1, H),
                     head=head)
    return h


# speedup vs baseline: 9.6674x; 1.4767x over previous
import functools

import jax
import jax.numpy as jnp
from jax import lax
from jax.experimental import pallas as pl
from jax.experimental.pallas import tpu as pltpu
from jax.experimental.pallas import tpu_sc as plsc

N = 10000
E = 320000
H = 128
G = 100
L = 2

NC = 2
NS = 16
NW = NC * NS
CH = 128
NCH = 80
EPW = CH * NCH
EPAD = EPW * NW
NP = 10112
RPS = NP // NS


def _sc_aggr_body(h_hbm, src_hbm, dst_hbm, z_hbm, out_hbm,
                  didx_v, sbuf0, sbuf1, rows0, rows1, shared_aggr,
                  isem0, isem1, gsem0, gsem1, ssem0, ssem1):
    c = lax.axis_index("c")
    s = lax.axis_index("s")
    wid = s * NC + c

    pltpu.sync_copy(z_hbm, shared_aggr.at[pl.ds(s * RPS, RPS)])
    pltpu.sync_copy(dst_hbm.at[wid], didx_v)
    plsc.subcore_barrier()

    def sidx(j, buf, sem):
        pltpu.async_copy(src_hbm.at[wid, j], buf, sem)

    def sidx_wait(buf, sem):
        pltpu.make_async_copy(src_hbm.at[wid, 0], buf, sem).wait()

    def gather(buf, idx, sem):
        pltpu.async_copy(h_hbm.at[idx], buf, sem)

    def gather_wait(buf, idx, sem):
        pltpu.make_async_copy(h_hbm.at[idx], buf, sem).wait()

    def scatter(j, buf, sem):
        pltpu.async_copy(buf, shared_aggr.at[didx_v.at[j]], sem, add=True)

    def scatter_wait(buf, sem):
        pltpu.make_async_copy(buf, shared_aggr.at[didx_v.at[0]], sem).wait()

    sidx(0, sbuf0, isem0)
    sidx(1, sbuf1, isem1)
    sidx_wait(sbuf0, isem0)
    gather(rows0, sbuf0, gsem0)

    def step(i, carry):
        j0 = 2 * i
        gather_wait(rows0, sbuf0, gsem0)

        @pl.when(j0 + 2 < NCH)
        def _():
            sidx(j0 + 2, sbuf0, isem0)

        scatter(j0, rows0, ssem0)

        @pl.when(i > 0)
        def _():
            scatter_wait(rows1, ssem1)

        sidx_wait(sbuf1, isem1)
        gather(rows1, sbuf1, gsem1)

        gather_wait(rows1, sbuf1, gsem1)

        @pl.when(j0 + 3 < NCH)
        def _():
            sidx(j0 + 3, sbuf1, isem1)

        scatter(j0 + 1, rows1, ssem1)
        scatter_wait(rows0, ssem0)

        @pl.when(j0 + 2 < NCH)
        def _():
            sidx_wait(sbuf0, isem0)
            gather(rows0, sbuf0, gsem0)

        return carry

    lax.fori_loop(0, NCH // 2, step, 0)
    scatter_wait(rows1, ssem1)
    plsc.subcore_barrier()

    pltpu.sync_copy(shared_aggr.at[pl.ds(s * RPS, RPS)],
                    out_hbm.at[c, pl.ds(s * RPS, RPS)])


def _sc_aggregate(h, src_r, dst_r, zeros):
    mesh = plsc.VectorSubcoreMesh(core_axis_name="c", subcore_axis_name="s")
    return pl.kernel(
        _sc_aggr_body,
        out_type=jax.ShapeDtypeStruct((NC, NP, H), jnp.float32),
        mesh=mesh,
        scratch_types=[
            pltpu.VMEM((NCH, CH), jnp.int32),
            pltpu.VMEM((CH,), jnp.int32),
            pltpu.VMEM((CH,), jnp.int32),
            pltpu.VMEM((CH, H), jnp.float32),
            pltpu.VMEM((CH, H), jnp.float32),
            pltpu.VMEM_SHARED((NP, H), jnp.float32),
            pltpu.SemaphoreType.DMA,
            pltpu.SemaphoreType.DMA,
            pltpu.SemaphoreType.DMA,
            pltpu.SemaphoreType.DMA,
            pltpu.SemaphoreType.DMA,
            pltpu.SemaphoreType.DMA,
        ],
        name="sc_edge_aggregate",
    )(h, src_r, dst_r, zeros)


def _combine_body(h_ref, p_ref, b_ref, vw_ref, vb_ref, aw_ref, ab_ref,
                  rw_ref, rb_ref, g_ref, bt_ref, *rest):
    if len(rest) == 3:
        pw_ref, pb_ref, o_ref = rest
        maybe_head = (pw_ref, pb_ref)
    else:
        (o_ref,) = rest
        maybe_head = None
    h = h_ref[:]
    aggr = p_ref[0, :N] + p_ref[1, :N]
    onehot = (b_ref[:] == lax.broadcasted_iota(jnp.int32, (1, H), 1)
              ).astype(jnp.float32)
    pooled = lax.dot_general(onehot, h, (((0,), (0,)), ((), ())),
                             preferred_element_type=jnp.float32)
    r_rot = jnp.dot(pooled, rw_ref[:], preferred_element_type=jnp.float32)
    r_term = jnp.dot(onehot, r_rot, preferred_element_type=jnp.float32)
    hn = (jnp.dot(h, vw_ref[:], preferred_element_type=jnp.float32)
          + jnp.dot(aggr, aw_ref[:], preferred_element_type=jnp.float32)
          + r_term + vb_ref[:] + ab_ref[:] + rb_ref[:])
    hn = jnp.maximum(hn, 0.0)
    mean = jnp.mean(hn, axis=0, keepdims=True)
    cen = hn - mean
    var = jnp.mean(cen * cen, axis=0, keepdims=True)
    out = cen * lax.rsqrt(var + 1e-5) * g_ref[:] + bt_ref[:]
    if maybe_head:
        pw_ref, pb_ref = maybe_head
        out = jnp.dot(out, pw_ref[:], preferred_element_type=jnp.float32) \
            + pb_ref[:]
    o_ref[:] = out


def _combine(h, parts, batch_col, vw, vb, aw, ab, rw, rb, g, bt,
             head=None):
    args = [h, parts, batch_col, vw, vb, aw, ab, rw, rb, g, bt]
    if head is not None:
        args += [head[0], head[1]]
    return pl.pallas_call(
        _combine_body,
        out_shape=jax.ShapeDtypeStruct((N, H), jnp.float32),
        name="tc_combine",
    )(*args)


@jax.jit
def kernel(x, edge_index, batch, Vw, Vb, Aw, Ab, Rw, Rb, gamma, beta, Pw, Pb):
    pad = EPAD - E
    pad_idx = jnp.arange(pad, dtype=jnp.int32)
    src_r = jnp.concatenate(
        [edge_index[0], (pad_idx * 13) % N]).reshape(NW, NCH, CH)
    dst_r = jnp.concatenate(
        [edge_index[1], N + pad_idx % (NP - N)]).reshape(NW, NCH, CH)
    zeros = jnp.zeros((RPS, H), dtype=jnp.float32)
    batch_col = batch.reshape(N, 1)

    h = x
    for l in range(L):
        parts = _sc_aggregate(h, src_r, dst_r, zeros)
        head = (Pw, Pb) if l == L - 1 else None
        h = _combine(h, parts, batch_col,
                     Vw[l], Vb[l].reshape(1, H), Aw[l], Ab[l].reshape(1, H),
                     Rw[l], Rb[l].reshape(1, H),
                     gamma[l].reshape(1, H), beta[l].reshape(1, H),
                     head=head)
    return h


# speedup vs baseline: 10.8977x; 1.1273x over previous
import functools

import jax
import jax.numpy as jnp
from jax import lax
from jax.experimental import pallas as pl
from jax.experimental.pallas import tpu as pltpu
from jax.experimental.pallas import tpu_sc as plsc

N = 10000
E = 320000
H = 128
G = 100
L = 2

NC = 2
NS = 16
NW = NC * NS
CH = 64
NCH = 159
EPW = CH * NCH
EPAD = EPW * NW
NP = 10112
RPS = NP // NS


DEPTH = 3


def _sc_aggr_body(h_hbm, src_hbm, dst_hbm, z_hbm, out_hbm,
                  didx_v, sbuf0, sbuf1, sbuf2, rows0, rows1, rows2,
                  shared_aggr, isem0, isem1, isem2,
                  gsem0, gsem1, gsem2, ssem0, ssem1, ssem2):
    c = lax.axis_index("c")
    s = lax.axis_index("s")
    wid = s * NC + c
    sbufs = (sbuf0, sbuf1, sbuf2)
    rows = (rows0, rows1, rows2)
    isems = (isem0, isem1, isem2)
    gsems = (gsem0, gsem1, gsem2)
    ssems = (ssem0, ssem1, ssem2)

    pltpu.sync_copy(z_hbm, shared_aggr.at[pl.ds(s * RPS, RPS)])
    pltpu.sync_copy(dst_hbm.at[wid], didx_v)
    plsc.subcore_barrier()

    def sidx(j, b):
        pltpu.async_copy(src_hbm.at[wid, j], sbufs[b], isems[b])

    def sidx_wait(b):
        pltpu.make_async_copy(src_hbm.at[wid, 0], sbufs[b], isems[b]).wait()

    def gather(b):
        pltpu.async_copy(h_hbm.at[sbufs[b]], rows[b], gsems[b])

    def gather_wait(b):
        pltpu.make_async_copy(h_hbm.at[sbufs[b]], rows[b], gsems[b]).wait()

    def scatter(j, b):
        pltpu.async_copy(rows[b], shared_aggr.at[didx_v.at[j]], ssems[b],
                         add=True)

    def scatter_wait(b):
        pltpu.make_async_copy(rows[b], shared_aggr.at[didx_v.at[0]],
                              ssems[b]).wait()

    for b in range(DEPTH):
        sidx(b, b)
    for b in range(DEPTH):
        sidx_wait(b)
        gather(b)

    def step(i, carry):
        j0 = i * DEPTH
        for b in range(DEPTH):
            j = j0 + b
            gather_wait(b)

            @pl.when(j + DEPTH < NCH)
            def _():
                sidx(j + DEPTH, b)

            scatter(j, b)

            @pl.when(j + DEPTH < NCH)
            def _():
                scatter_wait(b)
                sidx_wait(b)
                gather(b)

        return carry

    lax.fori_loop(0, NCH // DEPTH, step, 0)
    for b in range(DEPTH):
        scatter_wait(b)
    plsc.subcore_barrier()

    pltpu.sync_copy(shared_aggr.at[pl.ds(s * RPS, RPS)],
                    out_hbm.at[c, pl.ds(s * RPS, RPS)])


def _sc_aggregate(h, src_r, dst_r, zeros):
    mesh = plsc.VectorSubcoreMesh(core_axis_name="c", subcore_axis_name="s")
    return pl.kernel(
        _sc_aggr_body,
        out_type=jax.ShapeDtypeStruct((NC, NP, H), jnp.float32),
        mesh=mesh,
        scratch_types=[
            pltpu.VMEM((NCH, CH), jnp.int32),
            pltpu.VMEM((CH,), jnp.int32),
            pltpu.VMEM((CH,), jnp.int32),
            pltpu.VMEM((CH,), jnp.int32),
            pltpu.VMEM((CH, H), jnp.float32),
            pltpu.VMEM((CH, H), jnp.float32),
            pltpu.VMEM((CH, H), jnp.float32),
            pltpu.VMEM_SHARED((NP, H), jnp.float32),
            pltpu.SemaphoreType.DMA,
            pltpu.SemaphoreType.DMA,
            pltpu.SemaphoreType.DMA,
            pltpu.SemaphoreType.DMA,
            pltpu.SemaphoreType.DMA,
            pltpu.SemaphoreType.DMA,
            pltpu.SemaphoreType.DMA,
            pltpu.SemaphoreType.DMA,
            pltpu.SemaphoreType.DMA,
        ],
        name="sc_edge_aggregate",
    )(h, src_r, dst_r, zeros)


def _combine_body(h_ref, p_ref, b_ref, vw_ref, vb_ref, aw_ref, ab_ref,
                  rw_ref, rb_ref, g_ref, bt_ref, *rest):
    if len(rest) == 3:
        pw_ref, pb_ref, o_ref = rest
        maybe_head = (pw_ref, pb_ref)
    else:
        (o_ref,) = rest
        maybe_head = None
    h = h_ref[:]
    aggr = p_ref[0, :N] + p_ref[1, :N]
    onehot = (b_ref[:] == lax.broadcasted_iota(jnp.int32, (1, H), 1)
              ).astype(jnp.float32)
    pooled = lax.dot_general(onehot, h, (((0,), (0,)), ((), ())),
                             preferred_element_type=jnp.float32)
    r_rot = jnp.dot(pooled, rw_ref[:], preferred_element_type=jnp.float32)
    r_term = jnp.dot(onehot, r_rot, preferred_element_type=jnp.float32)
    hn = (jnp.dot(h, vw_ref[:], preferred_element_type=jnp.float32)
          + jnp.dot(aggr, aw_ref[:], preferred_element_type=jnp.float32)
          + r_term + vb_ref[:] + ab_ref[:] + rb_ref[:])
    hn = jnp.maximum(hn, 0.0)
    mean = jnp.mean(hn, axis=0, keepdims=True)
    cen = hn - mean
    var = jnp.mean(cen * cen, axis=0, keepdims=True)
    out = cen * lax.rsqrt(var + 1e-5) * g_ref[:] + bt_ref[:]
    if maybe_head:
        pw_ref, pb_ref = maybe_head
        out = jnp.dot(out, pw_ref[:], preferred_element_type=jnp.float32) \
            + pb_ref[:]
    o_ref[:] = out


def _combine(h, parts, batch_col, vw, vb, aw, ab, rw, rb, g, bt,
             head=None):
    args = [h, parts, batch_col, vw, vb, aw, ab, rw, rb, g, bt]
    if head is not None:
        args += [head[0], head[1]]
    return pl.pallas_call(
        _combine_body,
        out_shape=jax.ShapeDtypeStruct((N, H), jnp.float32),
        name="tc_combine",
    )(*args)


@jax.jit
def kernel(x, edge_index, batch, Vw, Vb, Aw, Ab, Rw, Rb, gamma, beta, Pw, Pb):
    pad = EPAD - E
    pad_idx = jnp.arange(pad, dtype=jnp.int32)
    src_r = jnp.concatenate(
        [edge_index[0], (pad_idx * 13) % N]).reshape(NW, NCH, CH)
    dst_r = jnp.concatenate(
        [edge_index[1], N + pad_idx % (NP - N)]).reshape(NW, NCH, CH)
    zeros = jnp.zeros((RPS, H), dtype=jnp.float32)
    batch_col = batch.reshape(N, 1)

    h = x
    for l in range(L):
        parts = _sc_aggregate(h, src_r, dst_r, zeros)
        head = (Pw, Pb) if l == L - 1 else None
        h = _combine(h, parts, batch_col,
                     Vw[l], Vb[l].reshape(1, H), Aw[l], Ab[l].reshape(1, H),
                     Rw[l], Rb[l].reshape(1, H),
                     gamma[l].reshape(1, H), beta[l].reshape(1, H),
                     head=head)
    return h
